# baseline (speedup 1.0000x reference)
import functools

import jax
import jax.numpy as jnp
from jax import lax
from jax.experimental import pallas as pl
from jax.experimental.pallas import tpu as pltpu
from jax.experimental.pallas import tpu_sc as plsc

NW = 32
CH = 128
TB = 256
D = 64


def _sc_feature_gather(tab_a, idx_a, stride_a, tab_b, idx_b, stride_b, sign):
    n = idx_a.shape[0]
    per_w = n // NW
    nch = per_w // CH
    mesh = plsc.VectorSubcoreMesh(core_axis_name="c", subcore_axis_name="s")

    scratch = [
        pltpu.VMEM((per_w,), jnp.int32),
        pltpu.VMEM((per_w,), jnp.int32),
        pltpu.VMEM((per_w,), jnp.int32),
        pltpu.VMEM((per_w,), jnp.int32),
        pltpu.VMEM((per_w,), jnp.float32),
        pltpu.VMEM((per_w,), jnp.float32),
        pltpu.VMEM((D, per_w), jnp.float32),
        pltpu.SemaphoreType.DMA,
    ]

    @functools.partial(
        pl.kernel,
        mesh=mesh,
        out_type=jax.ShapeDtypeStruct((D, n), jnp.float32),
        compiler_params=pltpu.CompilerParams(use_tc_tiling_on_sc=False),
        scratch_types=scratch,
    )
    def k(ta, tb, ia_hbm, ib_hbm, out_hbm, ia, ib, fa, fb, ga, gb, res, sem):
        wid = lax.axis_index("s") * 2 + lax.axis_index("c")
        base = wid * per_w
        pltpu.sync_copy(ia_hbm.at[pl.ds(base, per_w)], ia)
        pltpu.sync_copy(ib_hbm.at[pl.ds(base, per_w)], ib)

        def per_d(d, carry):
            offa = d * stride_a
            offb = d * stride_b
            for c in range(per_w // 16):
                s = pl.ds(c * 16, 16)
                fa[s] = ia[s] + offa
                fb[s] = ib[s] + offb
            copies = []
            for j in range(nch):
                s = pl.ds(j * CH, CH)
                copies.append(pltpu.async_copy(ta.at[fa.at[s]], ga.at[s], sem))
                copies.append(pltpu.async_copy(tb.at[fb.at[s]], gb.at[s], sem))
            for cp in copies:
                cp.wait()
            for c in range(per_w // 16):
                s = pl.ds(c * 16, 16)
                res[d, s] = ga[s] + sign * gb[s]
            return carry

        lax.fori_loop(0, D, per_d, 0, unroll=False)
        pltpu.sync_copy(res, out_hbm.at[:, pl.ds(base, per_w)])

    return k(tab_a, tab_b, idx_a, idx_b)


def _tc_segment_matmul(bounds, r_s3, vt, tm, nt):
    def body(bounds_ref, r_ref, vt_ref, tm_ref, out_ref):
        i = pl.program_id(0)
        lo = bounds_ref[i, 0]
        hi = bounds_ref[i, 1]
        v = vt_ref[...]
        rt = r_ref[0]

        def step(rr, acc):
            m = tm_ref[rr]
            prod = lax.dot_general(
                m, v, (((1,), (0,)), ((), ())),
                preferred_element_type=jnp.float32,
            )
            return acc + jnp.where(rt == rr, prod, 0.0)

        out_ref[...] = lax.fori_loop(
            lo, hi + 1, step, jnp.zeros((D, TB), jnp.float32))

    grid_spec = pltpu.PrefetchScalarGridSpec(
        num_scalar_prefetch=1,
        grid=(nt,),
        in_specs=[
            pl.BlockSpec((1, 1, TB), lambda i, s: (i, 0, 0)),
            pl.BlockSpec((D, TB), lambda i, s: (0, i)),
            pl.BlockSpec((tm.shape[0], D, D), lambda i, s: (0, 0, 0)),
        ],
        out_specs=pl.BlockSpec((D, TB), lambda i, s: (0, i)),
    )
    return pl.pallas_call(
        body,
        grid_spec=grid_spec,
        out_shape=jax.ShapeDtypeStruct((D, nt * TB), jnp.float32),
    )(bounds, r_s3, vt, tm)


def kernel(h, t, r, ent_embeddings, rel_embeddings, transfer_matrix):
    b = h.shape[0]
    e = ent_embeddings.shape[0]
    nr = rel_embeddings.shape[0]
    nt = b // TB
    h = h.astype(jnp.int32)
    t = t.astype(jnp.int32)
    r = r.astype(jnp.int32)

    order = jnp.argsort(r)
    r_s = jnp.take(r, order)
    h_s = jnp.take(h, order)
    t_s = jnp.take(t, order)
    inv = jnp.zeros((b,), jnp.int32).at[order].set(
        jnp.arange(b, dtype=jnp.int32))
    bounds = jnp.stack([r_s[::TB], r_s[TB - 1::TB]], axis=1)

    ent_flat = jnp.transpose(ent_embeddings.reshape(e, D)).reshape(e * D)
    rel_flat = jnp.transpose(rel_embeddings.reshape(nr, D)).reshape(nr * D)
    tm = transfer_matrix

    vt = _sc_feature_gather(ent_flat, h_s, e, ent_flat, t_s, e, -1.0)

    out_t = _tc_segment_matmul(bounds, r_s.reshape(nt, 1, TB), vt, tm, nt)

    out_p = _sc_feature_gather(
        out_t.reshape(D * b), inv, b, rel_flat, r, nr, 1.0)
    return jnp.transpose(out_p)


# speedup vs baseline: 5.9643x; 5.9643x over previous
import functools

import jax
import jax.numpy as jnp
from jax import lax
from jax.experimental import pallas as pl
from jax.experimental.pallas import tpu as pltpu
from jax.experimental.pallas import tpu_sc as plsc

NW = 32
CH = 128
TB = 256
D = 64


def _sc_row_gather(tab_a, idx_a, tab_b, idx_b, sign):
    n = idx_a.shape[0]
    per_w = n // NW
    nch = per_w // CH
    mesh = plsc.VectorSubcoreMesh(core_axis_name="c", subcore_axis_name="s")

    scratch = [
        pltpu.VMEM((per_w,), jnp.int32),
        pltpu.VMEM((per_w,), jnp.int32),
        pltpu.VMEM((per_w, D), jnp.float32),
        pltpu.VMEM((per_w, D), jnp.float32),
        pltpu.SemaphoreType.DMA,
    ]

    @functools.partial(
        pl.kernel,
        mesh=mesh,
        out_type=jax.ShapeDtypeStruct((n, D), jnp.float32),
        compiler_params=pltpu.CompilerParams(use_tc_tiling_on_sc=False),
        scratch_types=scratch,
    )
    def k(ta, tb, ia_hbm, ib_hbm, out_hbm, ia, ib, ga, gb, sem):
        wid = lax.axis_index("s") * 2 + lax.axis_index("c")
        base = wid * per_w
        pltpu.sync_copy(ia_hbm.at[pl.ds(base, per_w)], ia)
        pltpu.sync_copy(ib_hbm.at[pl.ds(base, per_w)], ib)
        copies = []
        for j in range(nch):
            s = pl.ds(j * CH, CH)
            copies.append(pltpu.async_copy(ta.at[ia.at[s]], ga.at[s], sem))
            copies.append(pltpu.async_copy(tb.at[ib.at[s]], gb.at[s], sem))
        for cp in copies:
            cp.wait()

        def per_row(i, carry):
            for c in range(D // 16):
                s = pl.ds(c * 16, 16)
                ga[i, s] = ga[i, s] + sign * gb[i, s]
            return carry

        lax.fori_loop(0, per_w, per_row, 0, unroll=False)
        pltpu.sync_copy(ga, out_hbm.at[pl.ds(base, per_w)])

    return k(tab_a, tab_b, idx_a, idx_b)


def _tc_segment_matmul(bounds, r_s3, v_s, tm, nt):
    def body(bounds_ref, r_ref, v_ref, tm_ref, out_ref):
        i = pl.program_id(0)
        lo = bounds_ref[i, 0]
        hi = bounds_ref[i, 1]
        v = v_ref[0]
        rt = r_ref[0]

        def step(rr, acc):
            m = tm_ref[rr]
            prod = lax.dot_general(
                v, m, (((1,), (1,)), ((), ())),
                preferred_element_type=jnp.float32,
            )
            return acc + jnp.where(rt == rr, prod, 0.0)

        out_ref[...] = lax.fori_loop(
            lo, hi + 1, step, jnp.zeros((TB, D), jnp.float32))

    grid_spec = pltpu.PrefetchScalarGridSpec(
        num_scalar_prefetch=1,
        grid=(nt,),
        in_specs=[
            pl.BlockSpec((1, TB, 1), lambda i, s: (i, 0, 0)),
            pl.BlockSpec((1, TB, D), lambda i, s: (i, 0, 0)),
            pl.BlockSpec((tm.shape[0], D, D), lambda i, s: (0, 0, 0)),
        ],
        out_specs=pl.BlockSpec((TB, D), lambda i, s: (i, 0)),
    )
    return pl.pallas_call(
        body,
        grid_spec=grid_spec,
        out_shape=jax.ShapeDtypeStruct((nt * TB, D), jnp.float32),
    )(bounds, r_s3, v_s, tm)


def kernel(h, t, r, ent_embeddings, rel_embeddings, transfer_matrix):
    b = h.shape[0]
    e = ent_embeddings.shape[0]
    nr = rel_embeddings.shape[0]
    nt = b // TB
    h = h.astype(jnp.int32)
    t = t.astype(jnp.int32)
    r = r.astype(jnp.int32)

    order = jnp.argsort(r)
    r_s = jnp.take(r, order)
    h_s = jnp.take(h, order)
    t_s = jnp.take(t, order)
    inv = jnp.zeros((b,), jnp.int32).at[order].set(
        jnp.arange(b, dtype=jnp.int32))
    bounds = jnp.stack([r_s[::TB], r_s[TB - 1::TB]], axis=1)

    ent = ent_embeddings.reshape(e, D)
    rel = rel_embeddings.reshape(nr, D)
    tm = transfer_matrix

    v_s = _sc_row_gather(ent, h_s, ent, t_s, -1.0)

    out_s = _tc_segment_matmul(
        bounds, r_s.reshape(nt, TB, 1), v_s.reshape(nt, TB, D), tm, nt)

    return _sc_row_gather(out_s, inv, rel, r, 1.0)


# speedup vs baseline: 6.1786x; 1.0359x over previous
import functools

import jax
import jax.numpy as jnp
from jax import lax
from jax.experimental import pallas as pl
from jax.experimental.pallas import tpu as pltpu
from jax.experimental.pallas import tpu_sc as plsc

NW = 32
CH = 128
TB = 256
D = 64


def _sc_pair_rows(tab2, idx_a, idx_b):
    n = idx_a.shape[0]
    per_w = n // NW
    nch = per_w // CH
    mesh = plsc.VectorSubcoreMesh(core_axis_name="c", subcore_axis_name="s")

    scratch = [
        pltpu.VMEM((per_w,), jnp.int32),
        pltpu.VMEM((per_w,), jnp.int32),
        pltpu.VMEM((per_w, 2 * D), jnp.float32),
        pltpu.SemaphoreType.DMA,
    ]

    @functools.partial(
        pl.kernel,
        mesh=mesh,
        out_type=[jax.ShapeDtypeStruct((n, 2 * D), jnp.float32),
                  jax.ShapeDtypeStruct((n, 2 * D), jnp.float32)],
        compiler_params=pltpu.CompilerParams(use_tc_tiling_on_sc=False),
        scratch_types=scratch,
    )
    def k(tab, ia_hbm, ib_hbm, oa_hbm, ob_hbm, ia, ib, gp, sem):
        wid = lax.axis_index("s") * 2 + lax.axis_index("c")
        base = wid * per_w
        pltpu.sync_copy(ia_hbm.at[pl.ds(base, per_w)], ia)
        pltpu.sync_copy(ib_hbm.at[pl.ds(base, per_w)], ib)
        for idx_v, out_hbm in ((ia, oa_hbm), (ib, ob_hbm)):
            copies = []
            for j in range(nch):
                s = pl.ds(j * CH, CH)
                copies.append(
                    pltpu.async_copy(tab.at[idx_v.at[s]], gp.at[s], sem))
            for cp in copies:
                cp.wait()
            pltpu.sync_copy(gp, out_hbm.at[pl.ds(base, per_w)])

    return k(tab2, idx_a, idx_b)


def _sc_row_gather(tab_a, idx_a, tab_b, idx_b, sign):
    n = idx_a.shape[0]
    per_w = n // NW
    nch = per_w // CH
    same = tab_b is None
    mesh = plsc.VectorSubcoreMesh(core_axis_name="c", subcore_axis_name="s")

    scratch = [
        pltpu.VMEM((per_w,), jnp.int32),
        pltpu.VMEM((per_w,), jnp.int32),
        pltpu.VMEM((per_w, D), jnp.float32),
        pltpu.VMEM((per_w, D), jnp.float32),
        pltpu.SemaphoreType.DMA,
    ]

    def body(ta, tb, ia_hbm, ib_hbm, out_hbm, ia, ib, ga, gb, sem):
        wid = lax.axis_index("s") * 2 + lax.axis_index("c")
        base = wid * per_w
        pltpu.sync_copy(ia_hbm.at[pl.ds(base, per_w)], ia)
        pltpu.sync_copy(ib_hbm.at[pl.ds(base, per_w)], ib)
        copies = []
        for j in range(nch):
            s = pl.ds(j * CH, CH)
            copies.append(pltpu.async_copy(ta.at[ia.at[s]], ga.at[s], sem))
            copies.append(pltpu.async_copy(tb.at[ib.at[s]], gb.at[s], sem))
        for cp in copies:
            cp.wait()

        def per_row(i, carry):
            for c in range(D // 16):
                s = pl.ds(c * 16, 16)
                ga[i, s] = ga[i, s] + sign * gb[i, s]
            return carry

        lax.fori_loop(0, per_w, per_row, 0, unroll=False)
        pltpu.sync_copy(ga, out_hbm.at[pl.ds(base, per_w)])

    kw = dict(
        mesh=mesh,
        out_type=jax.ShapeDtypeStruct((n, D), jnp.float32),
        compiler_params=pltpu.CompilerParams(use_tc_tiling_on_sc=False),
        scratch_types=scratch,
    )
    if same:
        def body1(ta, ia_hbm, ib_hbm, out_hbm, *rest):
            return body(ta, ta, ia_hbm, ib_hbm, out_hbm, *rest)
        return pl.kernel(body1, **kw)(tab_a, idx_a, idx_b)
    return pl.kernel(body, **kw)(tab_a, tab_b, idx_a, idx_b)


def _tc_segment_matmul(bounds, r_s3, hp3, tp3, pp3, tm, nt):
    def body(bounds_ref, r_ref, hp_ref, tp_ref, pp_ref, tm_ref, out_ref):
        i = pl.program_id(0)
        lo = bounds_ref[i, 0]
        hi = bounds_ref[i, 1]
        hp = hp_ref[0]
        tp = tp_ref[0]
        pp = pp_ref[0]
        vh = jnp.where(pp[:, 0:1] == 1, hp[:, D:2 * D], hp[:, 0:D])
        vt = jnp.where(pp[:, 1:2] == 1, tp[:, D:2 * D], tp[:, 0:D])
        v = vh - vt
        rt = r_ref[0]

        nr = tm_ref.shape[0]

        def step2(i, acc):
            rr = lo + 2 * i
            rr2 = jnp.minimum(rr + 1, nr - 1)
            m1 = tm_ref[rr]
            m2 = tm_ref[rr2]
            p1 = lax.dot_general(
                v, m1, (((1,), (1,)), ((), ())),
                preferred_element_type=jnp.float32)
            p2 = lax.dot_general(
                v, m2, (((1,), (1,)), ((), ())),
                preferred_element_type=jnp.float32)
            acc = acc + jnp.where(rt == rr, p1, 0.0)
            return acc + jnp.where((rt == rr + 1) & (rr + 1 <= hi), p2, 0.0)

        out_ref[...] = lax.fori_loop(
            0, (hi - lo) // 2 + 1, step2, jnp.zeros((TB, D), jnp.float32))

    grid_spec = pltpu.PrefetchScalarGridSpec(
        num_scalar_prefetch=1,
        grid=(nt,),
        in_specs=[
            pl.BlockSpec((1, TB, 1), lambda i, s: (i, 0, 0)),
            pl.BlockSpec((1, TB, 2 * D), lambda i, s: (i, 0, 0)),
            pl.BlockSpec((1, TB, 2 * D), lambda i, s: (i, 0, 0)),
            pl.BlockSpec((1, TB, 2), lambda i, s: (i, 0, 0)),
            pl.BlockSpec((tm.shape[0], D, D), lambda i, s: (0, 0, 0)),
        ],
        out_specs=pl.BlockSpec((TB, D), lambda i, s: (i, 0)),
    )
    return pl.pallas_call(
        body,
        grid_spec=grid_spec,
        out_shape=jax.ShapeDtypeStruct((nt * TB, D), jnp.float32),
    )(bounds, r_s3, hp3, tp3, pp3, tm)


def kernel(h, t, r, ent_embeddings, rel_embeddings, transfer_matrix):
    b = h.shape[0]
    e = ent_embeddings.shape[0]
    nr = rel_embeddings.shape[0]
    nt = b // TB
    h = h.astype(jnp.int32)
    t = t.astype(jnp.int32)
    r = r.astype(jnp.int32)

    shift = (b - 1).bit_length()
    keys = (r << shift) | jnp.arange(b, dtype=jnp.int32)
    skeys = lax.sort([keys], is_stable=False)[0]
    order = skeys & (b - 1)
    r_s = skeys >> shift
    h_s = jnp.take(h, order)
    t_s = jnp.take(t, order)
    inv = jnp.zeros((b,), jnp.int32).at[order].set(
        jnp.arange(b, dtype=jnp.int32))
    bounds = jnp.stack([r_s[::TB], r_s[TB - 1::TB]], axis=1)

    ent2 = ent_embeddings.reshape(e // 2, 2 * D)
    rel = rel_embeddings.reshape(nr, D)
    tm = transfer_matrix

    hp, tp = _sc_pair_rows(ent2, h_s >> 1, t_s >> 1)
    pp = jnp.stack([h_s & 1, t_s & 1], axis=1)

    out_s = _tc_segment_matmul(
        bounds, r_s.reshape(nt, TB, 1), hp.reshape(nt, TB, 2 * D),
        tp.reshape(nt, TB, 2 * D), pp.reshape(nt, TB, 2), tm, nt)

    return _sc_row_gather(out_s, inv, rel, r, 1.0)


# speedup vs baseline: 6.9708x; 1.1282x over previous
import functools

import jax
import jax.numpy as jnp
from jax import lax
from jax.experimental import pallas as pl
from jax.experimental.pallas import tpu as pltpu
from jax.experimental.pallas import tpu_sc as plsc

NW = 32
CH = 128
TB = 256
D = 64


def _sc_pad_gather_sub(tab128, idx_a, idx_b):
    n = idx_a.shape[0]
    per_w = n // NW
    nch = per_w // CH
    mesh = plsc.VectorSubcoreMesh(core_axis_name="c", subcore_axis_name="s")

    scratch = [
        pltpu.VMEM((per_w,), jnp.int32),
        pltpu.VMEM((per_w,), jnp.int32),
        pltpu.VMEM((per_w, 2 * D), jnp.float32),
        pltpu.VMEM((per_w, D), jnp.float32),
        pltpu.SemaphoreType.DMA,
    ]

    @functools.partial(
        pl.kernel,
        mesh=mesh,
        out_type=jax.ShapeDtypeStruct((n, D), jnp.float32),
        compiler_params=pltpu.CompilerParams(use_tc_tiling_on_sc=False),
        scratch_types=scratch,
    )
    def k(tab, ia_hbm, ib_hbm, out_hbm, ia, ib, gp, res, sem):
        wid = lax.axis_index("s") * 2 + lax.axis_index("c")
        base = wid * per_w
        pltpu.sync_copy(ia_hbm.at[pl.ds(base, per_w)], ia)
        pltpu.sync_copy(ib_hbm.at[pl.ds(base, per_w)], ib)

        def phase(idx_v):
            copies = []
            for j in range(nch):
                s = pl.ds(j * CH, CH)
                copies.append(
                    pltpu.async_copy(tab.at[idx_v.at[s]], gp.at[s], sem))
            for cp in copies:
                cp.wait()

        phase(ia)

        def rowcopy(i, carry):
            for c in range(D // 16):
                s = pl.ds(c * 16, 16)
                res[i, s] = gp[i, s]
            return carry

        lax.fori_loop(0, per_w, rowcopy, 0, unroll=False)
        phase(ib)

        def rowsub(i, carry):
            for c in range(D // 16):
                s = pl.ds(c * 16, 16)
                res[i, s] = res[i, s] - gp[i, s]
            return carry

        lax.fori_loop(0, per_w, rowsub, 0, unroll=False)
        pltpu.sync_copy(res, out_hbm.at[pl.ds(base, per_w)])

    return k(tab128, idx_a, idx_b)


def _sc_row_gather(tab_a, idx_a, tab_b, idx_b, sign):
    n = idx_a.shape[0]
    per_w = n // NW
    nch = per_w // CH
    same = tab_b is None
    mesh = plsc.VectorSubcoreMesh(core_axis_name="c", subcore_axis_name="s")

    scratch = [
        pltpu.VMEM((per_w,), jnp.int32),
        pltpu.VMEM((per_w,), jnp.int32),
        pltpu.VMEM((per_w, D), jnp.float32),
        pltpu.VMEM((per_w, D), jnp.float32),
        pltpu.SemaphoreType.DMA,
    ]

    def body(ta, tb, ia_hbm, ib_hbm, out_hbm, ia, ib, ga, gb, sem):
        wid = lax.axis_index("s") * 2 + lax.axis_index("c")
        base = wid * per_w
        pltpu.sync_copy(ia_hbm.at[pl.ds(base, per_w)], ia)
        pltpu.sync_copy(ib_hbm.at[pl.ds(base, per_w)], ib)
        copies = []
        for j in range(nch):
            s = pl.ds(j * CH, CH)
            copies.append(pltpu.async_copy(ta.at[ia.at[s]], ga.at[s], sem))
            copies.append(pltpu.async_copy(tb.at[ib.at[s]], gb.at[s], sem))
        for cp in copies:
            cp.wait()

        def per_row(i, carry):
            for c in range(D // 16):
                s = pl.ds(c * 16, 16)
                ga[i, s] = ga[i, s] + sign * gb[i, s]
            return carry

        lax.fori_loop(0, per_w, per_row, 0, unroll=False)
        pltpu.sync_copy(ga, out_hbm.at[pl.ds(base, per_w)])

    kw = dict(
        mesh=mesh,
        out_type=jax.ShapeDtypeStruct((n, D), jnp.float32),
        compiler_params=pltpu.CompilerParams(use_tc_tiling_on_sc=False),
        scratch_types=scratch,
    )
    if same:
        def body1(ta, ia_hbm, ib_hbm, out_hbm, *rest):
            return body(ta, ta, ia_hbm, ib_hbm, out_hbm, *rest)
        return pl.kernel(body1, **kw)(tab_a, idx_a, idx_b)
    return pl.kernel(body, **kw)(tab_a, tab_b, idx_a, idx_b)


def _tc_segment_matmul(bounds, r_s3, v_s, tm, nt):
    def body(bounds_ref, r_ref, v_ref, tm_ref, out_ref):
        i = pl.program_id(0)
        lo = bounds_ref[i, 0]
        hi = bounds_ref[i, 1]
        v = v_ref[0]
        rt = r_ref[0]

        nr = tm_ref.shape[0]

        def step2(i, acc):
            rr = lo + 2 * i
            rr2 = jnp.minimum(rr + 1, nr - 1)
            m1 = tm_ref[rr]
            m2 = tm_ref[rr2]
            p1 = lax.dot_general(
                v, m1, (((1,), (1,)), ((), ())),
                preferred_element_type=jnp.float32)
            p2 = lax.dot_general(
                v, m2, (((1,), (1,)), ((), ())),
                preferred_element_type=jnp.float32)
            acc = acc + jnp.where(rt == rr, p1, 0.0)
            return acc + jnp.where((rt == rr + 1) & (rr + 1 <= hi), p2, 0.0)

        out_ref[...] = lax.fori_loop(
            0, (hi - lo) // 2 + 1, step2, jnp.zeros((TB, D), jnp.float32))

    grid_spec = pltpu.PrefetchScalarGridSpec(
        num_scalar_prefetch=1,
        grid=(nt,),
        in_specs=[
            pl.BlockSpec((1, TB, 1), lambda i, s: (i, 0, 0)),
            pl.BlockSpec((1, TB, D), lambda i, s: (i, 0, 0)),
            pl.BlockSpec((tm.shape[0], D, D), lambda i, s: (0, 0, 0)),
        ],
        out_specs=pl.BlockSpec((TB, D), lambda i, s: (i, 0)),
    )
    return pl.pallas_call(
        body,
        grid_spec=grid_spec,
        out_shape=jax.ShapeDtypeStruct((nt * TB, D), jnp.float32),
    )(bounds, r_s3, v_s, tm)


def kernel(h, t, r, ent_embeddings, rel_embeddings, transfer_matrix):
    b = h.shape[0]
    e = ent_embeddings.shape[0]
    nr = rel_embeddings.shape[0]
    nt = b // TB
    h = h.astype(jnp.int32)
    t = t.astype(jnp.int32)
    r = r.astype(jnp.int32)

    shift = (b - 1).bit_length()
    keys = (r << shift) | jnp.arange(b, dtype=jnp.int32)
    skeys = lax.sort([keys], is_stable=False)[0]
    order = skeys & (b - 1)
    r_s = skeys >> shift
    h_s = jnp.take(h, order)
    t_s = jnp.take(t, order)
    inv = jnp.zeros((b,), jnp.int32).at[order].set(
        jnp.arange(b, dtype=jnp.int32))
    bounds = jnp.stack([r_s[::TB], r_s[TB - 1::TB]], axis=1)

    ent128 = jnp.pad(ent_embeddings.reshape(e, D), ((0, 0), (0, D)))
    rel = rel_embeddings.reshape(nr, D)
    tm = transfer_matrix

    v_s = _sc_pad_gather_sub(ent128, h_s, t_s)

    out_s = _tc_segment_matmul(
        bounds, r_s.reshape(nt, TB, 1), v_s.reshape(nt, TB, D), tm, nt)

    return _sc_row_gather(out_s, inv, rel, r, 1.0)
